# Initial kernel scaffold; baseline (speedup 1.0000x reference)
import jax
import jax.numpy as jnp
from jax.experimental import pallas as pl


def kernel(x, edge_attr, edge_index, batch, params):
    raise NotImplementedError("write your pallas kernel here")


# speedup vs baseline: 3.3792x; 3.3792x over previous
import functools

import jax
import jax.numpy as jnp
from jax import lax
from jax.experimental import pallas as pl
from jax.experimental.pallas import tpu as pltpu
from jax.experimental.pallas import tpu_sc as plsc

N = 10000
NP = 10240
E = 320000
DN = 128
DE = 16
H = 64

HP = 128

NC = 2
NS = 16
NW = NC * NS
EW = E // NW
NCHUNK = 10
CH = EW // NCHUNK
NSUB = 8
SUB = CH // NSUB
SCHUNK = 50
SCH = EW // SCHUNK
SNSUB = 2
SSUB = SCH // SNSUB
RPT = NP // NS

_mesh = plsc.VectorSubcoreMesh(core_axis_name="c", subcore_axis_name="s")


@functools.partial(
    pl.kernel,
    mesh=_mesh,
    out_type=jax.ShapeDtypeStruct((E, HP), jnp.float32),
    scratch_types=[
        pltpu.VMEM((NSUB, SUB), jnp.int32),
        pltpu.VMEM((CH, HP), jnp.float32),
        pltpu.SemaphoreType.DMA,
    ],
)
def _sc_gather(table_hbm, src_hbm, out_hbm, idx_v, rows_v, sem):
    cid = lax.axis_index("c")
    sid = lax.axis_index("s")
    wid = sid * NC + cid
    base = wid * EW
    for c in range(NCHUNK):
        pltpu.sync_copy(src_hbm.at[wid, c], idx_v)
        copies = []
        for j in range(NSUB):
            copies.append(
                pltpu.async_copy(
                    table_hbm.at[idx_v.at[j]],
                    rows_v.at[pl.ds(j * SUB, SUB)],
                    sem,
                )
            )
        for cp in copies:
            cp.wait()
        pltpu.sync_copy(rows_v, out_hbm.at[pl.ds(base + c * CH, CH)])


@functools.partial(
    pl.kernel,
    mesh=_mesh,
    out_type=jax.ShapeDtypeStruct((NC, NP, HP), jnp.float32),
    scratch_types=[
        pltpu.VMEM((SNSUB, SSUB), jnp.int32),
        pltpu.VMEM((SCH, HP), jnp.float32),
        pltpu.VMEM_SHARED((NP, HP), jnp.float32),
    ],
)
def _sc_scatter(m_hbm, dst_hbm, zeros_hbm, out_hbm, idx_v, rows_v, acc_sh):
    cid = lax.axis_index("c")
    sid = lax.axis_index("s")
    wid = sid * NC + cid
    pltpu.sync_copy(zeros_hbm.at[pl.ds(sid * RPT, RPT)],
                    acc_sh.at[pl.ds(sid * RPT, RPT)])
    plsc.subcore_barrier()
    base = wid * EW
    def chunk(c, _):
        pltpu.sync_copy(dst_hbm.at[wid, c], idx_v)
        pltpu.sync_copy(m_hbm.at[pl.ds(base + c * SCH, SCH)], rows_v)
        for j in range(SNSUB):
            pltpu.sync_copy(rows_v.at[pl.ds(j * SSUB, SSUB)],
                            acc_sh.at[idx_v.at[j]], add=True)
        return 0
    lax.fori_loop(0, SCHUNK, chunk, 0)
    plsc.subcore_barrier()
    pltpu.sync_copy(acc_sh.at[pl.ds(sid * RPT, RPT)],
                    out_hbm.at[cid, pl.ds(sid * RPT, RPT)])


def _tc_in_body(x_ref, w1_ref, b1_ref, w2_ref, b2_ref, w1a_ref,
                h_ref, hw_ref):
    h0 = jnp.maximum(x_ref[...] @ w1_ref[...] + b1_ref[...], 0.0)
    h = jnp.maximum(h0 @ w2_ref[...] + b2_ref[...], 0.0)
    h_ref[...] = h
    hw_ref[...] = h @ w1a_ref[...]


def _tc_input(x, w1, b1, w2, b2, w1a0):
    return pl.pallas_call(
        _tc_in_body,
        out_shape=(
            jax.ShapeDtypeStruct((N, H), jnp.float32),
            jax.ShapeDtypeStruct((N, HP), jnp.float32),
        ),
    )(x, w1, b1, w2, b2, w1a0)


BE = 8000


def _tc_edge_body(xjw_ref, ea_ref, w1b_ref, b1_ref, w2_ref, b2_ref, m_ref):
    t = jnp.maximum(xjw_ref[...][:, :H] + ea_ref[...] @ w1b_ref[...]
                    + b1_ref[...], 0.0)
    m_ref[...] = jnp.maximum(t @ w2_ref[...] + b2_ref[...], 0.0)


def _tc_edge(xjw, ea, w1b, b1, w2, b2):
    return pl.pallas_call(
        _tc_edge_body,
        grid=(E // BE,),
        in_specs=[
            pl.BlockSpec((BE, HP), lambda i: (i, 0)),
            pl.BlockSpec((BE, DE), lambda i: (i, 0)),
            pl.BlockSpec((DE, H), lambda i: (0, 0)),
            pl.BlockSpec((1, H), lambda i: (0, 0)),
            pl.BlockSpec((H, HP), lambda i: (0, 0)),
            pl.BlockSpec((1, HP), lambda i: (0, 0)),
        ],
        out_specs=pl.BlockSpec((BE, HP), lambda i: (i, 0)),
        out_shape=jax.ShapeDtypeStruct((E, HP), jnp.float32),
    )(xjw, ea, w1b, b1, w2, b2)


def _node_common(h_ref, p_ref, w3a_ref, w3b_ref, b3_ref, w4_ref,
                 b4_ref, g_ref, bb_ref):
    h = h_ref[...]
    p = p_ref[...]
    den = jnp.maximum(p[0, :N, H:H + 1] + p[1, :N, H:H + 1], 1.0)
    agg = (p[0, :N, :H] + p[1, :N, :H]) / den
    u = jnp.maximum(h @ w3a_ref[...] + agg @ w3b_ref[...] + b3_ref[...], 0.0)
    s = jax.nn.sigmoid(u @ w4_ref[...] + b4_ref[...])
    z = h + s
    mean = jnp.mean(z, axis=0, keepdims=True)
    var = jnp.mean((z - mean) ** 2, axis=0, keepdims=True)
    return (z - mean) / jnp.sqrt(var + 1e-5) * g_ref[...] + bb_ref[...]


def _tc_node_mid_body(h_ref, p_ref, w3a_ref, w3b_ref, b3_ref,
                      w4_ref, b4_ref, g_ref, bb_ref, w1a_ref,
                      h_out, hw_out):
    hn = _node_common(h_ref, p_ref, w3a_ref, w3b_ref, b3_ref,
                      w4_ref, b4_ref, g_ref, bb_ref)
    h_out[...] = hn
    hw_out[...] = hn @ w1a_ref[...]


def _tc_node_mid(h, parts, w3a, w3b, b3, w4, b4, g, bb, w1a_next):
    return pl.pallas_call(
        _tc_node_mid_body,
        out_shape=(
            jax.ShapeDtypeStruct((N, H), jnp.float32),
            jax.ShapeDtypeStruct((N, HP), jnp.float32),
        ),
    )(h, parts, w3a, w3b, b3, w4, b4, g, bb, w1a_next)


def _tc_node_last_body(h_ref, p_ref, w3a_ref, w3b_ref, b3_ref,
                       w4_ref, b4_ref, g_ref, bb_ref, f1_ref, c1_ref,
                       f2_ref, c2_ref, o_out):
    hn = _node_common(h_ref, p_ref, w3a_ref, w3b_ref, b3_ref,
                      w4_ref, b4_ref, g_ref, bb_ref)
    t = jnp.maximum(hn @ f1_ref[...] + c1_ref[...], 0.0)
    o_out[...] = jax.nn.sigmoid(t @ f2_ref[...] + c2_ref[...])


def _tc_node_last(h, parts, w3a, w3b, b3, w4, b4, g, bb, f1, c1, f2, c2):
    return pl.pallas_call(
        _tc_node_last_body,
        out_shape=jax.ShapeDtypeStruct((N, 1), jnp.float32),
    )(h, parts, w3a, w3b, b3, w4, b4, g, bb, f1, c1, f2, c2)


def kernel(x, edge_attr, edge_index, batch, params):
    src = edge_index[0].reshape(NW, NCHUNK, NSUB, SUB)
    dst = edge_index[1].reshape(NW, SCHUNK, SNSUB, SSUB)

    def row(b):
        return b.reshape(1, -1)

    layers = params["layers"]
    w1a = [jnp.pad(lp["m1_1"][0][:H], ((0, 0), (0, HP - H))) for lp in layers]
    w1b = [lp["m1_1"][0][H:] for lp in layers]
    b1 = [row(lp["m1_1"][1]) for lp in layers]
    w2 = [jnp.pad(lp["m1_2"][0], ((0, 0), (0, HP - H))) for lp in layers]
    cpad = jnp.zeros((1, HP - H), jnp.float32).at[0, 0].set(1.0)
    b2 = [jnp.concatenate([row(lp["m1_2"][1]), cpad], axis=1) for lp in layers]
    w3a = [lp["m2_1"][0][:H] for lp in layers]
    w3b = [lp["m2_1"][0][H:] for lp in layers]
    b3 = [row(lp["m2_1"][1]) for lp in layers]
    w4 = [lp["m2_2"][0] for lp in layers]
    b4 = [row(lp["m2_2"][1]) for lp in layers]
    g = [row(lp["bn_g"]) for lp in layers]
    bb = [row(lp["bn_b"]) for lp in layers]

    zeros_h = jnp.zeros((NP, HP), jnp.float32)

    h, hw = _tc_input(x, params["in1"][0], row(params["in1"][1]),
                      params["in2"][0], row(params["in2"][1]), w1a[0])

    for l in range(len(layers)):
        xjw = _sc_gather(hw, src)
        m = _tc_edge(xjw, edge_attr, w1b[l], b1[l], w2[l], b2[l])
        parts = _sc_scatter(m, dst, zeros_h)
        if l + 1 < len(layers):
            h, hw = _tc_node_mid(h, parts, w3a[l], w3b[l], b3[l],
                                 w4[l], b4[l], g[l], bb[l], w1a[l + 1])
        else:
            o = _tc_node_last(h, parts, w3a[l], w3b[l], b3[l],
                              w4[l], b4[l], g[l], bb[l],
                              params["f1"][0], row(params["f1"][1]),
                              params["f2"][0], row(params["f2"][1]))
    return o


# speedup vs baseline: 3.9384x; 1.1655x over previous
import functools

import jax
import jax.numpy as jnp
from jax import lax
from jax.experimental import pallas as pl
from jax.experimental.pallas import tpu as pltpu
from jax.experimental.pallas import tpu_sc as plsc

N = 10000
NP = 10240
E = 320000
DN = 128
DE = 16
H = 64

HP = 128

NC = 2
NS = 16
NW = NC * NS
EW = E // NW
GNCH = 25
GCH = EW // GNCH
GNSUB = 4
GSUB = GCH // GNSUB
SPH = 2
SNCH = 125
SCH = EW // (SPH * SNCH)
NBUF = 5
RPT = NP // NS

_mesh = plsc.VectorSubcoreMesh(core_axis_name="c", subcore_axis_name="s")


@functools.partial(
    pl.kernel,
    mesh=_mesh,
    out_type=jax.ShapeDtypeStruct((E, HP), jnp.float32),
    scratch_types=[
        pltpu.VMEM((GNCH, GNSUB, GSUB), jnp.int32),
        pltpu.VMEM((2, GCH, HP), jnp.float32),
        pltpu.SemaphoreType.DMA,
        pltpu.SemaphoreType.DMA,
    ],
)
def _sc_gather(table_hbm, src_hbm, out_hbm, idx_v, rows_v, sem_g, sem_o):
    cid = lax.axis_index("c")
    sid = lax.axis_index("s")
    wid = sid * NC + cid
    base = wid * EW
    pltpu.sync_copy(src_hbm.at[wid], idx_v)
    out_pending = [None, None]
    for c in range(GNCH):
        b = c % 2
        if out_pending[b] is not None:
            out_pending[b].wait()
        gs = []
        for j in range(GNSUB):
            gs.append(
                pltpu.async_copy(
                    table_hbm.at[idx_v.at[c, j]],
                    rows_v.at[b, pl.ds(j * GSUB, GSUB)],
                    sem_g,
                )
            )
        for x in gs:
            x.wait()
        out_pending[b] = pltpu.async_copy(
            rows_v.at[b], out_hbm.at[pl.ds(base + c * GCH, GCH)], sem_o)
    for h in out_pending:
        h.wait()


@functools.partial(
    pl.kernel,
    mesh=_mesh,
    out_type=jax.ShapeDtypeStruct((NC, NP, HP), jnp.float32),
    scratch_types=[
        pltpu.VMEM((SNCH, SCH), jnp.int32),
        pltpu.VMEM((NBUF, SCH, HP), jnp.float32),
        pltpu.VMEM_SHARED((NP, HP), jnp.float32),
        pltpu.SemaphoreType.DMA,
        pltpu.SemaphoreType.DMA,
    ],
)
def _sc_scatter(m_hbm, dst_hbm, zeros_hbm, out_hbm, idx_v, rows_v, acc_sh,
                sem_st, sem_ad):
    cid = lax.axis_index("c")
    sid = lax.axis_index("s")
    wid = sid * NC + cid
    pltpu.sync_copy(zeros_hbm.at[pl.ds(sid * RPT, RPT)],
                    acc_sh.at[pl.ds(sid * RPT, RPT)])
    plsc.subcore_barrier()

    def stage(pbase, c, buf):
        return pltpu.async_copy(
            m_hbm.at[pl.ds(pbase + c * SCH, SCH)], rows_v.at[buf], sem_st)

    def drain(sem):
        pltpu.make_async_copy(
            m_hbm.at[pl.ds(0, SCH)], rows_v.at[0], sem).wait()

    for p in range(SPH):
        pltpu.sync_copy(dst_hbm.at[wid, p], idx_v)
        pbase = wid * EW + p * SNCH * SCH
        stage(pbase, 0, 0)
        stage(pbase, 1, 1)

        def chunk(c, _):
            @pl.when(c >= 3)
            def _():
                drain(sem_ad)

            @pl.when(c + 2 < SNCH)
            def _():
                stage(pbase, c + 2, lax.rem(c + 2, NBUF))
            drain(sem_st)
            pltpu.async_copy(rows_v.at[lax.rem(c, NBUF)],
                             acc_sh.at[idx_v.at[c]], sem_ad, add=True)
            return 0
        lax.fori_loop(0, SNCH, chunk, 0)
        drain(sem_ad)
        drain(sem_ad)
        drain(sem_ad)
    plsc.subcore_barrier()
    pltpu.sync_copy(acc_sh.at[pl.ds(sid * RPT, RPT)],
                    out_hbm.at[cid, pl.ds(sid * RPT, RPT)])


def _tc_in_body(x_ref, w1_ref, b1_ref, w2_ref, b2_ref, w1a_ref,
                h_ref, hw_ref):
    h0 = jnp.maximum(x_ref[...] @ w1_ref[...] + b1_ref[...], 0.0)
    h = jnp.maximum(h0 @ w2_ref[...] + b2_ref[...], 0.0)
    h_ref[...] = h
    hw_ref[...] = h @ w1a_ref[...]


def _tc_input(x, w1, b1, w2, b2, w1a0):
    return pl.pallas_call(
        _tc_in_body,
        out_shape=(
            jax.ShapeDtypeStruct((N, H), jnp.float32),
            jax.ShapeDtypeStruct((N, HP), jnp.float32),
        ),
    )(x, w1, b1, w2, b2, w1a0)


BE = 8000


def _tc_edge_body(xjw_ref, ea_ref, w1b_ref, b1_ref, w2_ref, b2_ref, m_ref):
    t = jnp.maximum(xjw_ref[...][:, :H] + ea_ref[...] @ w1b_ref[...]
                    + b1_ref[...], 0.0)
    m_ref[...] = jnp.maximum(t @ w2_ref[...] + b2_ref[...], 0.0)


def _tc_edge(xjw, ea, w1b, b1, w2, b2):
    return pl.pallas_call(
        _tc_edge_body,
        grid=(E // BE,),
        in_specs=[
            pl.BlockSpec((BE, HP), lambda i: (i, 0)),
            pl.BlockSpec((BE, DE), lambda i: (i, 0)),
            pl.BlockSpec((DE, H), lambda i: (0, 0)),
            pl.BlockSpec((1, H), lambda i: (0, 0)),
            pl.BlockSpec((H, HP), lambda i: (0, 0)),
            pl.BlockSpec((1, HP), lambda i: (0, 0)),
        ],
        out_specs=pl.BlockSpec((BE, HP), lambda i: (i, 0)),
        out_shape=jax.ShapeDtypeStruct((E, HP), jnp.float32),
    )(xjw, ea, w1b, b1, w2, b2)


def _node_common(h_ref, p_ref, w3a_ref, w3b_ref, b3_ref, w4_ref,
                 b4_ref, g_ref, bb_ref):
    h = h_ref[...]
    p = p_ref[...]
    den = jnp.maximum(p[0, :N, H:H + 1] + p[1, :N, H:H + 1], 1.0)
    agg = (p[0, :N, :H] + p[1, :N, :H]) / den
    u = jnp.maximum(h @ w3a_ref[...] + agg @ w3b_ref[...] + b3_ref[...], 0.0)
    s = jax.nn.sigmoid(u @ w4_ref[...] + b4_ref[...])
    z = h + s
    mean = jnp.mean(z, axis=0, keepdims=True)
    var = jnp.mean((z - mean) ** 2, axis=0, keepdims=True)
    return (z - mean) / jnp.sqrt(var + 1e-5) * g_ref[...] + bb_ref[...]


def _tc_node_mid_body(h_ref, p_ref, w3a_ref, w3b_ref, b3_ref,
                      w4_ref, b4_ref, g_ref, bb_ref, w1a_ref,
                      h_out, hw_out):
    hn = _node_common(h_ref, p_ref, w3a_ref, w3b_ref, b3_ref,
                      w4_ref, b4_ref, g_ref, bb_ref)
    h_out[...] = hn
    hw_out[...] = hn @ w1a_ref[...]


def _tc_node_mid(h, parts, w3a, w3b, b3, w4, b4, g, bb, w1a_next):
    return pl.pallas_call(
        _tc_node_mid_body,
        out_shape=(
            jax.ShapeDtypeStruct((N, H), jnp.float32),
            jax.ShapeDtypeStruct((N, HP), jnp.float32),
        ),
    )(h, parts, w3a, w3b, b3, w4, b4, g, bb, w1a_next)


def _tc_node_last_body(h_ref, p_ref, w3a_ref, w3b_ref, b3_ref,
                       w4_ref, b4_ref, g_ref, bb_ref, f1_ref, c1_ref,
                       f2_ref, c2_ref, o_out):
    hn = _node_common(h_ref, p_ref, w3a_ref, w3b_ref, b3_ref,
                      w4_ref, b4_ref, g_ref, bb_ref)
    t = jnp.maximum(hn @ f1_ref[...] + c1_ref[...], 0.0)
    o_out[...] = jax.nn.sigmoid(t @ f2_ref[...] + c2_ref[...])


def _tc_node_last(h, parts, w3a, w3b, b3, w4, b4, g, bb, f1, c1, f2, c2):
    return pl.pallas_call(
        _tc_node_last_body,
        out_shape=jax.ShapeDtypeStruct((N, 1), jnp.float32),
    )(h, parts, w3a, w3b, b3, w4, b4, g, bb, f1, c1, f2, c2)


def kernel(x, edge_attr, edge_index, batch, params):
    src = edge_index[0].reshape(NW, GNCH, GNSUB, GSUB)
    dst = edge_index[1].reshape(NW, SPH, SNCH, SCH)

    def row(b):
        return b.reshape(1, -1)

    layers = params["layers"]
    w1a = [jnp.pad(lp["m1_1"][0][:H], ((0, 0), (0, HP - H))) for lp in layers]
    w1b = [lp["m1_1"][0][H:] for lp in layers]
    b1 = [row(lp["m1_1"][1]) for lp in layers]
    w2 = [jnp.pad(lp["m1_2"][0], ((0, 0), (0, HP - H))) for lp in layers]
    cpad = jnp.zeros((1, HP - H), jnp.float32).at[0, 0].set(1.0)
    b2 = [jnp.concatenate([row(lp["m1_2"][1]), cpad], axis=1) for lp in layers]
    w3a = [lp["m2_1"][0][:H] for lp in layers]
    w3b = [lp["m2_1"][0][H:] for lp in layers]
    b3 = [row(lp["m2_1"][1]) for lp in layers]
    w4 = [lp["m2_2"][0] for lp in layers]
    b4 = [row(lp["m2_2"][1]) for lp in layers]
    g = [row(lp["bn_g"]) for lp in layers]
    bb = [row(lp["bn_b"]) for lp in layers]

    zeros_h = jnp.zeros((NP, HP), jnp.float32)

    h, hw = _tc_input(x, params["in1"][0], row(params["in1"][1]),
                      params["in2"][0], row(params["in2"][1]), w1a[0])

    for l in range(len(layers)):
        xjw = _sc_gather(hw, src)
        m = _tc_edge(xjw, edge_attr, w1b[l], b1[l], w2[l], b2[l])
        parts = _sc_scatter(m, dst, zeros_h)
        if l + 1 < len(layers):
            h, hw = _tc_node_mid(h, parts, w3a[l], w3b[l], b3[l],
                                 w4[l], b4[l], g[l], bb[l], w1a[l + 1])
        else:
            o = _tc_node_last(h, parts, w3a[l], w3b[l], b3[l],
                              w4[l], b4[l], g[l], bb[l],
                              params["f1"][0], row(params["f1"][1]),
                              params["f2"][0], row(params["f2"][1]))
    return o


# speedup vs baseline: 3.9674x; 1.0074x over previous
import functools

import jax
import jax.numpy as jnp
from jax import lax
from jax.experimental import pallas as pl
from jax.experimental.pallas import tpu as pltpu
from jax.experimental.pallas import tpu_sc as plsc

N = 10000
NP = 10240
E = 320000
DN = 128
DE = 16
H = 64

HP = 128

NC = 2
NS = 16
NW = NC * NS
EW = E // NW
E2 = E // 2
EW2 = E2 // NW
GNCH = 25
GCH = EW2 // GNCH
GNSUB = 2
GSUB = GCH // GNSUB
SNCH = 125
SCH = EW2 // SNCH
NBUF = 5
RPT = NP // NS

_mesh = plsc.VectorSubcoreMesh(core_axis_name="c", subcore_axis_name="s")


@functools.partial(
    pl.kernel,
    mesh=_mesh,
    out_type=jax.ShapeDtypeStruct((E2, HP), jnp.float32),
    scratch_types=[
        pltpu.VMEM((GNCH, GNSUB, GSUB), jnp.int32),
        pltpu.VMEM((2, GCH, HP), jnp.float32),
        pltpu.SemaphoreType.DMA,
        pltpu.SemaphoreType.DMA,
    ],
)
def _sc_gather(table_hbm, src_hbm, out_hbm, idx_v, rows_v, sem_g, sem_o):
    cid = lax.axis_index("c")
    sid = lax.axis_index("s")
    wid = sid * NC + cid
    base = wid * EW2
    pltpu.sync_copy(src_hbm.at[wid], idx_v)
    out_pending = [None, None]
    for c in range(GNCH):
        b = c % 2
        if out_pending[b] is not None:
            out_pending[b].wait()
        gs = []
        for j in range(GNSUB):
            gs.append(
                pltpu.async_copy(
                    table_hbm.at[idx_v.at[c, j]],
                    rows_v.at[b, pl.ds(j * GSUB, GSUB)],
                    sem_g,
                )
            )
        for x in gs:
            x.wait()
        out_pending[b] = pltpu.async_copy(
            rows_v.at[b], out_hbm.at[pl.ds(base + c * GCH, GCH)], sem_o)
    for h in out_pending:
        h.wait()


@functools.partial(
    pl.kernel,
    mesh=_mesh,
    out_type=jax.ShapeDtypeStruct((NC, NP, HP), jnp.float32),
    scratch_types=[
        pltpu.VMEM((SNCH, SCH), jnp.int32),
        pltpu.VMEM((NBUF, SCH, HP), jnp.float32),
        pltpu.VMEM_SHARED((NP, HP), jnp.float32),
        pltpu.SemaphoreType.DMA,
        pltpu.SemaphoreType.DMA,
    ],
)
def _sc_scatter(m_hbm, dst_hbm, zeros_hbm, out_hbm, idx_v, rows_v, acc_sh,
                sem_st, sem_ad):
    cid = lax.axis_index("c")
    sid = lax.axis_index("s")
    wid = sid * NC + cid
    pltpu.sync_copy(zeros_hbm.at[pl.ds(sid * RPT, RPT)],
                    acc_sh.at[pl.ds(sid * RPT, RPT)])
    plsc.subcore_barrier()

    def stage(pbase, c, buf):
        return pltpu.async_copy(
            m_hbm.at[pl.ds(pbase + c * SCH, SCH)], rows_v.at[buf], sem_st)

    def drain(sem):
        pltpu.make_async_copy(
            m_hbm.at[pl.ds(0, SCH)], rows_v.at[0], sem).wait()

    pltpu.sync_copy(dst_hbm.at[wid], idx_v)
    pbase = wid * EW2
    stage(pbase, 0, 0)
    stage(pbase, 1, 1)

    def chunk(c, _):
        @pl.when(c >= 3)
        def _():
            drain(sem_ad)

        @pl.when(c + 2 < SNCH)
        def _():
            stage(pbase, c + 2, lax.rem(c + 2, NBUF))
        drain(sem_st)
        pltpu.async_copy(rows_v.at[lax.rem(c, NBUF)],
                         acc_sh.at[idx_v.at[c]], sem_ad, add=True)
        return 0
    lax.fori_loop(0, SNCH, chunk, 0)
    drain(sem_ad)
    drain(sem_ad)
    drain(sem_ad)
    plsc.subcore_barrier()
    pltpu.sync_copy(acc_sh.at[pl.ds(sid * RPT, RPT)],
                    out_hbm.at[cid, pl.ds(sid * RPT, RPT)])


def _tc_in_body(x_ref, w1_ref, b1_ref, w2_ref, b2_ref, w1a_ref,
                h_ref, hw_ref):
    h0 = jnp.maximum(x_ref[...] @ w1_ref[...] + b1_ref[...], 0.0)
    h = jnp.maximum(h0 @ w2_ref[...] + b2_ref[...], 0.0)
    h_ref[...] = h
    hw_ref[...] = h @ w1a_ref[...]


def _tc_input(x, w1, b1, w2, b2, w1a0):
    return pl.pallas_call(
        _tc_in_body,
        out_shape=(
            jax.ShapeDtypeStruct((N, H), jnp.float32),
            jax.ShapeDtypeStruct((N, HP), jnp.float32),
        ),
    )(x, w1, b1, w2, b2, w1a0)


BE = 8000


def _tc_edge_body(xjw_ref, ea_ref, w1b_ref, b1_ref, w2_ref, b2_ref, m_ref):
    t = jnp.maximum(xjw_ref[...][:, :H] + ea_ref[...] @ w1b_ref[...]
                    + b1_ref[...], 0.0)
    m_ref[...] = jnp.maximum(t @ w2_ref[...] + b2_ref[...], 0.0)


def _tc_edge(h, xjw, ea, w1b, b1, w2, b2):
    nblk = E2 // BE
    off = h * nblk

    return pl.pallas_call(
        _tc_edge_body,
        grid=(nblk,),
        in_specs=[
            pl.BlockSpec((BE, HP), lambda i: (i, 0)),
            pl.BlockSpec((BE, DE), lambda i: (i + off, 0)),
            pl.BlockSpec((DE, H), lambda i: (0, 0)),
            pl.BlockSpec((1, H), lambda i: (0, 0)),
            pl.BlockSpec((H, HP), lambda i: (0, 0)),
            pl.BlockSpec((1, HP), lambda i: (0, 0)),
        ],
        out_specs=pl.BlockSpec((BE, HP), lambda i: (i, 0)),
        out_shape=jax.ShapeDtypeStruct((E2, HP), jnp.float32),
    )(xjw, ea, w1b, b1, w2, b2)


def _node_common(h_ref, p_ref, q_ref, w3a_ref, w3b_ref, b3_ref, w4_ref,
                 b4_ref, g_ref, bb_ref):
    h = h_ref[...]
    p = p_ref[...] + q_ref[...]
    ps = p[0, :N] + p[1, :N]
    den = jnp.maximum(ps[:, H:H + 1], 1.0)
    agg = ps[:, :H] / den
    u = jnp.maximum(h @ w3a_ref[...] + agg @ w3b_ref[...] + b3_ref[...], 0.0)
    s = jax.nn.sigmoid(u @ w4_ref[...] + b4_ref[...])
    z = h + s
    mean = jnp.mean(z, axis=0, keepdims=True)
    var = jnp.mean((z - mean) ** 2, axis=0, keepdims=True)
    return (z - mean) / jnp.sqrt(var + 1e-5) * g_ref[...] + bb_ref[...]


def _tc_node_mid_body(h_ref, p_ref, q_ref, w3a_ref, w3b_ref, b3_ref,
                      w4_ref, b4_ref, g_ref, bb_ref, w1a_ref,
                      h_out, hw_out):
    hn = _node_common(h_ref, p_ref, q_ref, w3a_ref, w3b_ref, b3_ref,
                      w4_ref, b4_ref, g_ref, bb_ref)
    h_out[...] = hn
    hw_out[...] = hn @ w1a_ref[...]


def _tc_node_mid(h, p0, p1, w3a, w3b, b3, w4, b4, g, bb, w1a_next):
    return pl.pallas_call(
        _tc_node_mid_body,
        out_shape=(
            jax.ShapeDtypeStruct((N, H), jnp.float32),
            jax.ShapeDtypeStruct((N, HP), jnp.float32),
        ),
    )(h, p0, p1, w3a, w3b, b3, w4, b4, g, bb, w1a_next)


def _tc_node_last_body(h_ref, p_ref, q_ref, w3a_ref, w3b_ref, b3_ref,
                       w4_ref, b4_ref, g_ref, bb_ref, f1_ref, c1_ref,
                       f2_ref, c2_ref, o_out):
    hn = _node_common(h_ref, p_ref, q_ref, w3a_ref, w3b_ref, b3_ref,
                      w4_ref, b4_ref, g_ref, bb_ref)
    t = jnp.maximum(hn @ f1_ref[...] + c1_ref[...], 0.0)
    o_out[...] = jax.nn.sigmoid(t @ f2_ref[...] + c2_ref[...])


def _tc_node_last(h, p0, p1, w3a, w3b, b3, w4, b4, g, bb, f1, c1, f2, c2):
    return pl.pallas_call(
        _tc_node_last_body,
        out_shape=jax.ShapeDtypeStruct((N, 1), jnp.float32),
    )(h, p0, p1, w3a, w3b, b3, w4, b4, g, bb, f1, c1, f2, c2)


def kernel(x, edge_attr, edge_index, batch, params):
    src0 = edge_index[0][:E2].reshape(NW, GNCH, GNSUB, GSUB)
    src1 = edge_index[0][E2:].reshape(NW, GNCH, GNSUB, GSUB)
    dst0 = edge_index[1][:E2].reshape(NW, SNCH, SCH)
    dst1 = edge_index[1][E2:].reshape(NW, SNCH, SCH)

    def row(b):
        return b.reshape(1, -1)

    layers = params["layers"]
    w1a = [jnp.pad(lp["m1_1"][0][:H], ((0, 0), (0, HP - H))) for lp in layers]
    w1b = [lp["m1_1"][0][H:] for lp in layers]
    b1 = [row(lp["m1_1"][1]) for lp in layers]
    w2 = [jnp.pad(lp["m1_2"][0], ((0, 0), (0, HP - H))) for lp in layers]
    cpad = jnp.zeros((1, HP - H), jnp.float32).at[0, 0].set(1.0)
    b2 = [jnp.concatenate([row(lp["m1_2"][1]), cpad], axis=1) for lp in layers]
    w3a = [lp["m2_1"][0][:H] for lp in layers]
    w3b = [lp["m2_1"][0][H:] for lp in layers]
    b3 = [row(lp["m2_1"][1]) for lp in layers]
    w4 = [lp["m2_2"][0] for lp in layers]
    b4 = [row(lp["m2_2"][1]) for lp in layers]
    g = [row(lp["bn_g"]) for lp in layers]
    bb = [row(lp["bn_b"]) for lp in layers]

    zeros_h = jnp.zeros((NP, HP), jnp.float32)

    h, hw = _tc_input(x, params["in1"][0], row(params["in1"][1]),
                      params["in2"][0], row(params["in2"][1]), w1a[0])

    for l in range(len(layers)):
        xj0 = _sc_gather(hw, src0)
        xj1 = _sc_gather(hw, src1)
        m0 = _tc_edge(0, xj0, edge_attr, w1b[l], b1[l], w2[l], b2[l])
        m1 = _tc_edge(1, xj1, edge_attr, w1b[l], b1[l], w2[l], b2[l])
        p0 = _sc_scatter(m0, dst0, zeros_h)
        p1 = _sc_scatter(m1, dst1, zeros_h)
        if l + 1 < len(layers):
            h, hw = _tc_node_mid(h, p0, p1, w3a[l], w3b[l], b3[l],
                                 w4[l], b4[l], g[l], bb[l], w1a[l + 1])
        else:
            o = _tc_node_last(h, p0, p1, w3a[l], w3b[l], b3[l],
                              w4[l], b4[l], g[l], bb[l],
                              params["f1"][0], row(params["f1"][1]),
                              params["f2"][0], row(params["f2"][1]))
    return o


# speedup vs baseline: 4.5541x; 1.1479x over previous
import functools

import jax
import jax.numpy as jnp
from jax import lax
from jax.experimental import pallas as pl
from jax.experimental.pallas import tpu as pltpu
from jax.experimental.pallas import tpu_sc as plsc

N = 10000
NP = 10240
E = 320000
DN = 128
DE = 16
H = 64

HP = 128

NC = 2
NS = 16
NW = NC * NS
EW = E // NW
E2 = E // 2
EW2 = E2 // NW
GNCH = 125
GCH = EW2 // GNCH
GSUB = GCH
GBUF = 4
SNCH = 125
SCH = EW2 // SNCH
NBUF = 5
RPT = NP // NS

_mesh = plsc.VectorSubcoreMesh(core_axis_name="c", subcore_axis_name="s")


@functools.partial(
    pl.kernel,
    mesh=_mesh,
    out_type=jax.ShapeDtypeStruct((E2, HP), jnp.float32),
    scratch_types=[
        pltpu.VMEM((GNCH, GSUB), jnp.int32),
        pltpu.VMEM((GBUF, GCH, HP), jnp.float32),
        pltpu.VMEM_SHARED((NP, HP), jnp.float32),
        pltpu.SemaphoreType.DMA,
        pltpu.SemaphoreType.DMA,
    ],
)
def _sc_gather(table_hbm, src_hbm, out_hbm, idx_v, rows_v, tab_sh,
               sem_g, sem_o):
    cid = lax.axis_index("c")
    sid = lax.axis_index("s")
    wid = sid * NC + cid
    base = wid * EW2
    pltpu.sync_copy(src_hbm.at[wid], idx_v)
    pltpu.sync_copy(table_hbm.at[pl.ds(sid * RPT, RPT)],
                    tab_sh.at[pl.ds(sid * RPT, RPT)])
    plsc.subcore_barrier()

    def gfire(c):
        pltpu.async_copy(tab_sh.at[idx_v.at[c]],
                         rows_v.at[lax.rem(c, GBUF)], sem_g)

    def drain(sem):
        pltpu.make_async_copy(
            out_hbm.at[pl.ds(0, GCH)], rows_v.at[0], sem).wait()

    gfire(0)
    gfire(1)

    def chunk(c, _):
        @pl.when(c >= 2)
        def _():
            drain(sem_o)

        @pl.when(c + 2 < GNCH)
        def _():
            gfire(c + 2)
        drain(sem_g)
        pltpu.async_copy(rows_v.at[lax.rem(c, GBUF)],
                         out_hbm.at[pl.ds(base + c * GCH, GCH)], sem_o)
        return 0
    lax.fori_loop(0, GNCH, chunk, 0)
    drain(sem_o)
    drain(sem_o)


@functools.partial(
    pl.kernel,
    mesh=_mesh,
    out_type=jax.ShapeDtypeStruct((NC, NP, HP), jnp.float32),
    scratch_types=[
        pltpu.VMEM((SNCH, SCH), jnp.int32),
        pltpu.VMEM((NBUF, SCH, HP), jnp.float32),
        pltpu.VMEM_SHARED((NP, HP), jnp.float32),
        pltpu.SemaphoreType.DMA,
        pltpu.SemaphoreType.DMA,
    ],
)
def _sc_scatter(m_hbm, dst_hbm, zeros_hbm, out_hbm, idx_v, rows_v, acc_sh,
                sem_st, sem_ad):
    cid = lax.axis_index("c")
    sid = lax.axis_index("s")
    wid = sid * NC + cid
    pltpu.sync_copy(zeros_hbm.at[pl.ds(sid * RPT, RPT)],
                    acc_sh.at[pl.ds(sid * RPT, RPT)])
    plsc.subcore_barrier()

    def stage(pbase, c, buf):
        return pltpu.async_copy(
            m_hbm.at[pl.ds(pbase + c * SCH, SCH)], rows_v.at[buf], sem_st)

    def drain(sem):
        pltpu.make_async_copy(
            m_hbm.at[pl.ds(0, SCH)], rows_v.at[0], sem).wait()

    pltpu.sync_copy(dst_hbm.at[wid], idx_v)
    pbase = wid * EW2
    stage(pbase, 0, 0)
    stage(pbase, 1, 1)

    def chunk(c, _):
        @pl.when(c >= 3)
        def _():
            drain(sem_ad)

        @pl.when(c + 2 < SNCH)
        def _():
            stage(pbase, c + 2, lax.rem(c + 2, NBUF))
        drain(sem_st)
        pltpu.async_copy(rows_v.at[lax.rem(c, NBUF)],
                         acc_sh.at[idx_v.at[c]], sem_ad, add=True)
        return 0
    lax.fori_loop(0, SNCH, chunk, 0)
    drain(sem_ad)
    drain(sem_ad)
    drain(sem_ad)
    plsc.subcore_barrier()
    pltpu.sync_copy(acc_sh.at[pl.ds(sid * RPT, RPT)],
                    out_hbm.at[cid, pl.ds(sid * RPT, RPT)])


def _tc_in_body(x_ref, w1_ref, b1_ref, w2_ref, b2_ref, w1a_ref,
                h_ref, hw_ref):
    h0 = jnp.maximum(x_ref[...] @ w1_ref[...] + b1_ref[...], 0.0)
    h = jnp.maximum(h0 @ w2_ref[...] + b2_ref[...], 0.0)
    h_ref[...] = h
    hw_ref[...] = h @ w1a_ref[...]


def _tc_input(x, w1, b1, w2, b2, w1a0):
    return pl.pallas_call(
        _tc_in_body,
        out_shape=(
            jax.ShapeDtypeStruct((N, H), jnp.float32),
            jax.ShapeDtypeStruct((N, HP), jnp.float32),
        ),
    )(x, w1, b1, w2, b2, w1a0)


BE = 8000


def _tc_edge_body(xjw_ref, ea_ref, w1b_ref, b1_ref, w2_ref, b2_ref, m_ref):
    t = jnp.maximum(xjw_ref[...][:, :H] + ea_ref[...] @ w1b_ref[...]
                    + b1_ref[...], 0.0)
    m_ref[...] = jnp.maximum(t @ w2_ref[...] + b2_ref[...], 0.0)


def _tc_edge(h, xjw, ea, w1b, b1, w2, b2):
    nblk = E2 // BE
    off = h * nblk

    return pl.pallas_call(
        _tc_edge_body,
        grid=(nblk,),
        in_specs=[
            pl.BlockSpec((BE, HP), lambda i: (i, 0)),
            pl.BlockSpec((BE, DE), lambda i: (i + off, 0)),
            pl.BlockSpec((DE, H), lambda i: (0, 0)),
            pl.BlockSpec((1, H), lambda i: (0, 0)),
            pl.BlockSpec((H, HP), lambda i: (0, 0)),
            pl.BlockSpec((1, HP), lambda i: (0, 0)),
        ],
        out_specs=pl.BlockSpec((BE, HP), lambda i: (i, 0)),
        out_shape=jax.ShapeDtypeStruct((E2, HP), jnp.float32),
    )(xjw, ea, w1b, b1, w2, b2)


def _node_common(h_ref, p_ref, q_ref, w3a_ref, w3b_ref, b3_ref, w4_ref,
                 b4_ref, g_ref, bb_ref):
    h = h_ref[...]
    p = p_ref[...] + q_ref[...]
    ps = p[0, :N] + p[1, :N]
    den = jnp.maximum(ps[:, H:H + 1], 1.0)
    agg = ps[:, :H] / den
    u = jnp.maximum(h @ w3a_ref[...] + agg @ w3b_ref[...] + b3_ref[...], 0.0)
    s = jax.nn.sigmoid(u @ w4_ref[...] + b4_ref[...])
    z = h + s
    mean = jnp.mean(z, axis=0, keepdims=True)
    var = jnp.mean((z - mean) ** 2, axis=0, keepdims=True)
    return (z - mean) / jnp.sqrt(var + 1e-5) * g_ref[...] + bb_ref[...]


def _tc_node_mid_body(h_ref, p_ref, q_ref, w3a_ref, w3b_ref, b3_ref,
                      w4_ref, b4_ref, g_ref, bb_ref, w1a_ref,
                      h_out, hw_out):
    hn = _node_common(h_ref, p_ref, q_ref, w3a_ref, w3b_ref, b3_ref,
                      w4_ref, b4_ref, g_ref, bb_ref)
    h_out[...] = hn
    hw_out[pl.ds(0, N), :] = hn @ w1a_ref[...]


def _tc_node_mid(h, p0, p1, w3a, w3b, b3, w4, b4, g, bb, w1a_next):
    return pl.pallas_call(
        _tc_node_mid_body,
        out_shape=(
            jax.ShapeDtypeStruct((N, H), jnp.float32),
            jax.ShapeDtypeStruct((NP, HP), jnp.float32),
        ),
    )(h, p0, p1, w3a, w3b, b3, w4, b4, g, bb, w1a_next)


def _tc_node_last_body(h_ref, p_ref, q_ref, w3a_ref, w3b_ref, b3_ref,
                       w4_ref, b4_ref, g_ref, bb_ref, f1_ref, c1_ref,
                       f2_ref, c2_ref, o_out):
    hn = _node_common(h_ref, p_ref, q_ref, w3a_ref, w3b_ref, b3_ref,
                      w4_ref, b4_ref, g_ref, bb_ref)
    t = jnp.maximum(hn @ f1_ref[...] + c1_ref[...], 0.0)
    o_out[...] = jax.nn.sigmoid(t @ f2_ref[...] + c2_ref[...])


def _tc_node_last(h, p0, p1, w3a, w3b, b3, w4, b4, g, bb, f1, c1, f2, c2):
    return pl.pallas_call(
        _tc_node_last_body,
        out_shape=jax.ShapeDtypeStruct((N, 1), jnp.float32),
    )(h, p0, p1, w3a, w3b, b3, w4, b4, g, bb, f1, c1, f2, c2)


def kernel(x, edge_attr, edge_index, batch, params):
    src0 = edge_index[0][:E2].reshape(NW, GNCH, GSUB)
    src1 = edge_index[0][E2:].reshape(NW, GNCH, GSUB)
    dst0 = edge_index[1][:E2].reshape(NW, SNCH, SCH)
    dst1 = edge_index[1][E2:].reshape(NW, SNCH, SCH)

    def row(b):
        return b.reshape(1, -1)

    layers = params["layers"]
    w1a = [jnp.pad(lp["m1_1"][0][:H], ((0, 0), (0, HP - H))) for lp in layers]
    w1b = [lp["m1_1"][0][H:] for lp in layers]
    b1 = [row(lp["m1_1"][1]) for lp in layers]
    w2 = [jnp.pad(lp["m1_2"][0], ((0, 0), (0, HP - H))) for lp in layers]
    cpad = jnp.zeros((1, HP - H), jnp.float32).at[0, 0].set(1.0)
    b2 = [jnp.concatenate([row(lp["m1_2"][1]), cpad], axis=1) for lp in layers]
    w3a = [lp["m2_1"][0][:H] for lp in layers]
    w3b = [lp["m2_1"][0][H:] for lp in layers]
    b3 = [row(lp["m2_1"][1]) for lp in layers]
    w4 = [lp["m2_2"][0] for lp in layers]
    b4 = [row(lp["m2_2"][1]) for lp in layers]
    g = [row(lp["bn_g"]) for lp in layers]
    bb = [row(lp["bn_b"]) for lp in layers]

    zeros_h = jnp.zeros((NP, HP), jnp.float32)

    h, hw = _tc_input(x, params["in1"][0], row(params["in1"][1]),
                      params["in2"][0], row(params["in2"][1]), w1a[0])

    for l in range(len(layers)):
        xj0 = _sc_gather(hw, src0)
        xj1 = _sc_gather(hw, src1)
        m0 = _tc_edge(0, xj0, edge_attr, w1b[l], b1[l], w2[l], b2[l])
        m1 = _tc_edge(1, xj1, edge_attr, w1b[l], b1[l], w2[l], b2[l])
        p0 = _sc_scatter(m0, dst0, zeros_h)
        p1 = _sc_scatter(m1, dst1, zeros_h)
        if l + 1 < len(layers):
            h, hw = _tc_node_mid(h, p0, p1, w3a[l], w3b[l], b3[l],
                                 w4[l], b4[l], g[l], bb[l], w1a[l + 1])
        else:
            o = _tc_node_last(h, p0, p1, w3a[l], w3b[l], b3[l],
                              w4[l], b4[l], g[l], bb[l],
                              params["f1"][0], row(params["f1"][1]),
                              params["f2"][0], row(params["f2"][1]))
    return o


# speedup vs baseline: 4.5577x; 1.0008x over previous
import functools

import jax
import jax.numpy as jnp
from jax import lax
from jax.experimental import pallas as pl
from jax.experimental.pallas import tpu as pltpu
from jax.experimental.pallas import tpu_sc as plsc

N = 10000
NP = 10240
E = 320000
DN = 128
DE = 16
H = 64

HP = 128

NC = 2
NS = 16
NW = NC * NS
EW = E // NW
E2 = E // 2
EW2 = E2 // NW
GNCH = 125
GCH = EW2 // GNCH
GSUB = GCH
GBUF = 6
SNCH = 125
SCH = EW2 // SNCH
NBUF = 6
RPT = NP // NS

_mesh = plsc.VectorSubcoreMesh(core_axis_name="c", subcore_axis_name="s")


@functools.partial(
    pl.kernel,
    mesh=_mesh,
    out_type=jax.ShapeDtypeStruct((E2, HP), jnp.float32),
    scratch_types=[
        pltpu.VMEM((GNCH, GSUB), jnp.int32),
        pltpu.VMEM((GBUF, GCH, HP), jnp.float32),
        pltpu.VMEM_SHARED((NP, HP), jnp.float32),
        pltpu.SemaphoreType.DMA,
        pltpu.SemaphoreType.DMA,
    ],
)
def _sc_gather(table_hbm, src_hbm, out_hbm, idx_v, rows_v, tab_sh,
               sem_g, sem_o):
    cid = lax.axis_index("c")
    sid = lax.axis_index("s")
    wid = sid * NC + cid
    base = wid * EW2
    pltpu.sync_copy(src_hbm.at[wid], idx_v)
    pltpu.sync_copy(table_hbm.at[pl.ds(sid * RPT, RPT)],
                    tab_sh.at[pl.ds(sid * RPT, RPT)])
    plsc.subcore_barrier()

    def gfire(c):
        pltpu.async_copy(tab_sh.at[idx_v.at[c]],
                         rows_v.at[lax.rem(c, GBUF)], sem_g)

    def drain(sem):
        pltpu.make_async_copy(
            out_hbm.at[pl.ds(0, GCH)], rows_v.at[0], sem).wait()

    gfire(0)
    gfire(1)

    def chunk(c, _):
        @pl.when(c >= 4)
        def _():
            drain(sem_o)

        @pl.when(c + 2 < GNCH)
        def _():
            gfire(c + 2)
        drain(sem_g)
        pltpu.async_copy(rows_v.at[lax.rem(c, GBUF)],
                         out_hbm.at[pl.ds(base + c * GCH, GCH)], sem_o)
        return 0
    lax.fori_loop(0, GNCH, chunk, 0)
    for _ in range(4):
        drain(sem_o)


@functools.partial(
    pl.kernel,
    mesh=_mesh,
    out_type=jax.ShapeDtypeStruct((NC, NP, HP), jnp.float32),
    scratch_types=[
        pltpu.VMEM((SNCH, SCH), jnp.int32),
        pltpu.VMEM((NBUF, SCH, HP), jnp.float32),
        pltpu.VMEM_SHARED((NP, HP), jnp.float32),
        pltpu.SemaphoreType.DMA,
        pltpu.SemaphoreType.DMA,
    ],
)
def _sc_scatter(m_hbm, dst_hbm, zeros_hbm, out_hbm, idx_v, rows_v, acc_sh,
                sem_st, sem_ad):
    cid = lax.axis_index("c")
    sid = lax.axis_index("s")
    wid = sid * NC + cid
    pltpu.sync_copy(zeros_hbm.at[pl.ds(sid * RPT, RPT)],
                    acc_sh.at[pl.ds(sid * RPT, RPT)])
    plsc.subcore_barrier()

    def stage(pbase, c, buf):
        return pltpu.async_copy(
            m_hbm.at[pl.ds(pbase + c * SCH, SCH)], rows_v.at[buf], sem_st)

    def drain(sem):
        pltpu.make_async_copy(
            m_hbm.at[pl.ds(0, SCH)], rows_v.at[0], sem).wait()

    pltpu.sync_copy(dst_hbm.at[wid], idx_v)
    pbase = wid * EW2
    stage(pbase, 0, 0)
    stage(pbase, 1, 1)

    def chunk(c, _):
        @pl.when(c >= 4)
        def _():
            drain(sem_ad)

        @pl.when(c + 2 < SNCH)
        def _():
            stage(pbase, c + 2, lax.rem(c + 2, NBUF))
        drain(sem_st)
        pltpu.async_copy(rows_v.at[lax.rem(c, NBUF)],
                         acc_sh.at[idx_v.at[c]], sem_ad, add=True)
        return 0
    lax.fori_loop(0, SNCH, chunk, 0)
    for _ in range(4):
        drain(sem_ad)
    plsc.subcore_barrier()
    pltpu.sync_copy(acc_sh.at[pl.ds(sid * RPT, RPT)],
                    out_hbm.at[cid, pl.ds(sid * RPT, RPT)])


def _tc_in_body(x_ref, w1_ref, b1_ref, w2_ref, b2_ref, w1a_ref,
                h_ref, hw_ref):
    h0 = jnp.maximum(x_ref[...] @ w1_ref[...] + b1_ref[...], 0.0)
    h = jnp.maximum(h0 @ w2_ref[...] + b2_ref[...], 0.0)
    h_ref[...] = h
    hw_ref[...] = h @ w1a_ref[...]


def _tc_input(x, w1, b1, w2, b2, w1a0):
    return pl.pallas_call(
        _tc_in_body,
        out_shape=(
            jax.ShapeDtypeStruct((N, H), jnp.float32),
            jax.ShapeDtypeStruct((N, HP), jnp.float32),
        ),
    )(x, w1, b1, w2, b2, w1a0)


BE = 8000


def _tc_edge_body(xjw_ref, ea_ref, w1b_ref, b1_ref, w2_ref, b2_ref, m_ref):
    t = jnp.maximum(xjw_ref[...][:, :H] + ea_ref[...] @ w1b_ref[...]
                    + b1_ref[...], 0.0)
    m_ref[...] = jnp.maximum(t @ w2_ref[...] + b2_ref[...], 0.0)


def _tc_edge(h, xjw, ea, w1b, b1, w2, b2):
    nblk = E2 // BE
    off = h * nblk

    return pl.pallas_call(
        _tc_edge_body,
        grid=(nblk,),
        in_specs=[
            pl.BlockSpec((BE, HP), lambda i: (i, 0)),
            pl.BlockSpec((BE, DE), lambda i: (i + off, 0)),
            pl.BlockSpec((DE, H), lambda i: (0, 0)),
            pl.BlockSpec((1, H), lambda i: (0, 0)),
            pl.BlockSpec((H, HP), lambda i: (0, 0)),
            pl.BlockSpec((1, HP), lambda i: (0, 0)),
        ],
        out_specs=pl.BlockSpec((BE, HP), lambda i: (i, 0)),
        out_shape=jax.ShapeDtypeStruct((E2, HP), jnp.float32),
    )(xjw, ea, w1b, b1, w2, b2)


def _node_common(h_ref, p_ref, q_ref, w3a_ref, w3b_ref, b3_ref, w4_ref,
                 b4_ref, g_ref, bb_ref):
    h = h_ref[...]
    p = p_ref[...] + q_ref[...]
    ps = p[0, :N] + p[1, :N]
    den = jnp.maximum(ps[:, H:H + 1], 1.0)
    agg = ps[:, :H] / den
    u = jnp.maximum(h @ w3a_ref[...] + agg @ w3b_ref[...] + b3_ref[...], 0.0)
    s = jax.nn.sigmoid(u @ w4_ref[...] + b4_ref[...])
    z = h + s
    mean = jnp.mean(z, axis=0, keepdims=True)
    var = jnp.mean((z - mean) ** 2, axis=0, keepdims=True)
    return (z - mean) / jnp.sqrt(var + 1e-5) * g_ref[...] + bb_ref[...]


def _tc_node_mid_body(h_ref, p_ref, q_ref, w3a_ref, w3b_ref, b3_ref,
                      w4_ref, b4_ref, g_ref, bb_ref, w1a_ref,
                      h_out, hw_out):
    hn = _node_common(h_ref, p_ref, q_ref, w3a_ref, w3b_ref, b3_ref,
                      w4_ref, b4_ref, g_ref, bb_ref)
    h_out[...] = hn
    hw_out[pl.ds(0, N), :] = hn @ w1a_ref[...]


def _tc_node_mid(h, p0, p1, w3a, w3b, b3, w4, b4, g, bb, w1a_next):
    return pl.pallas_call(
        _tc_node_mid_body,
        out_shape=(
            jax.ShapeDtypeStruct((N, H), jnp.float32),
            jax.ShapeDtypeStruct((NP, HP), jnp.float32),
        ),
    )(h, p0, p1, w3a, w3b, b3, w4, b4, g, bb, w1a_next)


def _tc_node_last_body(h_ref, p_ref, q_ref, w3a_ref, w3b_ref, b3_ref,
                       w4_ref, b4_ref, g_ref, bb_ref, f1_ref, c1_ref,
                       f2_ref, c2_ref, o_out):
    hn = _node_common(h_ref, p_ref, q_ref, w3a_ref, w3b_ref, b3_ref,
                      w4_ref, b4_ref, g_ref, bb_ref)
    t = jnp.maximum(hn @ f1_ref[...] + c1_ref[...], 0.0)
    o_out[...] = jax.nn.sigmoid(t @ f2_ref[...] + c2_ref[...])


def _tc_node_last(h, p0, p1, w3a, w3b, b3, w4, b4, g, bb, f1, c1, f2, c2):
    return pl.pallas_call(
        _tc_node_last_body,
        out_shape=jax.ShapeDtypeStruct((N, 1), jnp.float32),
    )(h, p0, p1, w3a, w3b, b3, w4, b4, g, bb, f1, c1, f2, c2)


def kernel(x, edge_attr, edge_index, batch, params):
    src0 = edge_index[0][:E2].reshape(NW, GNCH, GSUB)
    src1 = edge_index[0][E2:].reshape(NW, GNCH, GSUB)
    dst0 = edge_index[1][:E2].reshape(NW, SNCH, SCH)
    dst1 = edge_index[1][E2:].reshape(NW, SNCH, SCH)

    def row(b):
        return b.reshape(1, -1)

    layers = params["layers"]
    w1a = [jnp.pad(lp["m1_1"][0][:H], ((0, 0), (0, HP - H))) for lp in layers]
    w1b = [lp["m1_1"][0][H:] for lp in layers]
    b1 = [row(lp["m1_1"][1]) for lp in layers]
    w2 = [jnp.pad(lp["m1_2"][0], ((0, 0), (0, HP - H))) for lp in layers]
    cpad = jnp.zeros((1, HP - H), jnp.float32).at[0, 0].set(1.0)
    b2 = [jnp.concatenate([row(lp["m1_2"][1]), cpad], axis=1) for lp in layers]
    w3a = [lp["m2_1"][0][:H] for lp in layers]
    w3b = [lp["m2_1"][0][H:] for lp in layers]
    b3 = [row(lp["m2_1"][1]) for lp in layers]
    w4 = [lp["m2_2"][0] for lp in layers]
    b4 = [row(lp["m2_2"][1]) for lp in layers]
    g = [row(lp["bn_g"]) for lp in layers]
    bb = [row(lp["bn_b"]) for lp in layers]

    zeros_h = jnp.zeros((NP, HP), jnp.float32)

    h, hw = _tc_input(x, params["in1"][0], row(params["in1"][1]),
                      params["in2"][0], row(params["in2"][1]), w1a[0])

    for l in range(len(layers)):
        xj0 = _sc_gather(hw, src0)
        xj1 = _sc_gather(hw, src1)
        m0 = _tc_edge(0, xj0, edge_attr, w1b[l], b1[l], w2[l], b2[l])
        m1 = _tc_edge(1, xj1, edge_attr, w1b[l], b1[l], w2[l], b2[l])
        p0 = _sc_scatter(m0, dst0, zeros_h)
        p1 = _sc_scatter(m1, dst1, zeros_h)
        if l + 1 < len(layers):
            h, hw = _tc_node_mid(h, p0, p1, w3a[l], w3b[l], b3[l],
                                 w4[l], b4[l], g[l], bb[l], w1a[l + 1])
        else:
            o = _tc_node_last(h, p0, p1, w3a[l], w3b[l], b3[l],
                              w4[l], b4[l], g[l], bb[l],
                              params["f1"][0], row(params["f1"][1]),
                              params["f2"][0], row(params["f2"][1]))
    return o
